# baseline (speedup 1.0000x reference)
import functools
import math

import jax
import jax.numpy as jnp
from jax import lax
from jax.experimental import pallas as pl
from jax.experimental.pallas import tpu as pltpu
from jax.experimental.pallas import tpu_sc as plsc

B = 16384
S_DIM = 32
T_DIM = 32
PI_REF = 3.1415926235897933
GAMMA = 12.0
EMB_RANGE = GAMMA / float(S_DIM + T_DIM)
INV_2SCALE = PI_REF / (2.0 * EMB_RANGE)

NW = 32
QPW = B // NW
CH = 128
NCHUNK = QPW // CH

_TWO_PI = 6.283185307179586
_INV_2PI = 0.15915494309189535
_C3 = -0.1666666587584901
_C5 = 0.00833332023467762
_C7 = -0.00019840491560017788
_C9 = 2.7535159818767513e-06
_C11 = -2.472396353305536e-08
_C13 = 1.3601221017511822e-10


def _sin(x):
    k = x * _INV_2PI
    k = k + 0.5 * jnp.sign(k)
    kf = lax.convert_element_type(lax.convert_element_type(k, jnp.int32), jnp.float32)
    r = x - kf * _TWO_PI
    r2 = r * r
    p = _C13
    p = p * r2 + _C11
    p = p * r2 + _C9
    p = p * r2 + _C7
    p = p * r2 + _C5
    p = p * r2 + _C3
    return r + r * (r2 * p)


def _sinc(z):
    s = z * math.pi
    return jnp.where(z == 0.0, jnp.float32(1.0), _sin(s) / s)


def _sqrt(a):
    i = lax.bitcast_convert_type(a, jnp.int32)
    i = 0x5F3759DF - lax.shift_right_logical(i, 1)
    y = lax.bitcast_convert_type(i, jnp.float32)
    for _ in range(3):
        y = y * (1.5 - 0.5 * a * y * y)
    return jnp.where(a > 0.0, a * y, jnp.float32(0.0))


def _make_kernel():
    mesh = plsc.VectorSubcoreMesh(core_axis_name="c", subcore_axis_name="s")
    f32 = jnp.float32

    scratch = (
        [pltpu.VMEM((QPW,), jnp.int32)] * 3
        + [pltpu.VMEM((QPW,), f32)] * 3
        + [pltpu.VMEM((CH, S_DIM), f32)] * 22
        + [pltpu.VMEM((CH, 2 * S_DIM), f32)] * 3
        + [pltpu.VMEM((CH * 16,), f32)] * 2
        + [pltpu.VMEM((QPW,), f32)]
        + [pltpu.SemaphoreType.DMA]
    )

    @functools.partial(
        pl.kernel,
        out_type=jax.ShapeDtypeStruct((B,), f32),
        mesh=mesh,
        scratch_types=scratch,
        compiler_params=pltpu.CompilerParams(
            needs_layout_passes=False, use_tc_tiling_on_sc=False),
    )
    def sc_kernel(heads, rels, tails, years, months, days,
                  ent_h, ent_t, rel_f, rel_i, rel_j,
                  m_fh, m_ft, m_ph, m_pt, m_ah, m_at,
                  d_fh, d_ft, d_ph, d_pt, d_ah, d_at,
                  y_fh, y_ft, y_ph, y_pt, y_ah, y_at,
                  out,
                  hv, rv, tv, yv, mv, dv,
                  g_eh_h, g_et_h, g_eh_t, g_et_t,
                  g_yfh, g_yph, g_yah, g_mfh, g_mph, g_mah,
                  g_dfh, g_dph, g_dah,
                  g_yft, g_ypt, g_yat, g_mft, g_mpt, g_mat,
                  g_dft, g_dpt, g_dat,
                  g_r1, g_r2, g_r3,
                  psb, msb, outv, sem):
        wid = lax.axis_index("s") * 2 + lax.axis_index("c")
        base = wid * QPW

        pltpu.sync_copy(heads.at[pl.ds(base, QPW)], hv)
        pltpu.sync_copy(rels.at[pl.ds(base, QPW)], rv)
        pltpu.sync_copy(tails.at[pl.ds(base, QPW)], tv)
        pltpu.sync_copy(years.at[pl.ds(base, QPW)], yv)
        pltpu.sync_copy(months.at[pl.ds(base, QPW)], mv)
        pltpu.sync_copy(days.at[pl.ds(base, QPW)], dv)

        def tbody(i, carry):
            s = pl.multiple_of(i * 16, 16)
            yv[pl.ds(s, 16)] = yv[pl.ds(s, 16)] - 2010.0
            mv[pl.ds(s, 16)] = mv[pl.ds(s, 16)] * (1.0 / 6.0) - 1.0
            dv[pl.ds(s, 16)] = dv[pl.ds(s, 16)] * 0.0625 - 1.0
            return carry
        lax.fori_loop(0, QPW // 16, tbody, 0)

        head_tabs = [(ent_h, g_eh_h), (ent_t, g_et_h),
                     (y_fh, g_yfh), (y_ph, g_yph), (y_ah, g_yah),
                     (m_fh, g_mfh), (m_ph, g_mph), (m_ah, g_mah),
                     (d_fh, g_dfh), (d_ph, g_dph), (d_ah, g_dah)]
        tail_tabs = [(ent_h, g_eh_t), (ent_t, g_et_t),
                     (y_ft, g_yft), (y_pt, g_ypt), (y_at, g_yat),
                     (m_ft, g_mft), (m_pt, g_mpt), (m_at, g_mat),
                     (d_ft, g_dft), (d_pt, g_dpt), (d_at, g_dat)]
        rel_tabs = [(rel_f, g_r1), (rel_i, g_r2), (rel_j, g_r3)]

        for c in range(NCHUNK):
            co = c * CH
            handles = []
            hidx = hv.at[pl.ds(co, CH)]
            tidx = tv.at[pl.ds(co, CH)]
            ridx = rv.at[pl.ds(co, CH)]
            for tab, buf in head_tabs:
                handles.append(pltpu.async_copy(tab.at[hidx], buf, sem))
            for tab, buf in tail_tabs:
                handles.append(pltpu.async_copy(tab.at[tidx], buf, sem))
            for tab, buf in rel_tabs:
                handles.append(pltpu.async_copy(tab.at[ridx], buf, sem))
            for h in handles:
                h.wait()

            def qbody(q, carry):
                qsplat = jnp.full((16,), co + q, jnp.int32)
                ty = plsc.load_gather(yv, [qsplat])
                tm = plsc.load_gather(mv, [qsplat])
                td = plsc.load_gather(dv, [qsplat])
                ps = jnp.zeros((16,), f32)
                ms = jnp.zeros((16,), f32)
                for c2 in range(2):
                    dsl = pl.ds(c2 * 16, 16)
                    th = (g_yah[q, dsl] * _sinc(g_yfh[q, dsl] * ty + g_yph[q, dsl])
                          + g_mah[q, dsl] * _sinc(g_mfh[q, dsl] * tm + g_mph[q, dsl])
                          + g_dah[q, dsl] * _sinc(g_dfh[q, dsl] * td + g_dph[q, dsl]))
                    tt = (g_yat[q, dsl] * _sinc(g_yft[q, dsl] * ty + g_ypt[q, dsl])
                          + g_mat[q, dsl] * _sinc(g_mft[q, dsl] * tm + g_mpt[q, dsl])
                          + g_dat[q, dsl] * _sinc(g_dft[q, dsl] * td + g_dpt[q, dsl]))
                    eh_h = g_eh_h[q, dsl]
                    et_h = g_et_h[q, dsl]
                    eh_t = g_eh_t[q, dsl]
                    et_t = g_et_t[q, dsl]
                    dsl2 = pl.ds(32 + c2 * 16, 16)
                    r1a = g_r1[q, dsl]
                    r1b = g_r1[q, dsl2]
                    phase1 = (eh_h + r1a - et_t) * INV_2SCALE
                    phase2 = (th + r1b - tt) * INV_2SCALE
                    ps = ps + jnp.abs(_sin(phase1)) + jnp.abs(_sin(phase2))
                    r2a = g_r2[q, dsl]
                    r3a = g_r3[q, dsl]
                    moda = jnp.abs(r2a)
                    biasa = jnp.maximum(jnp.minimum(r3a, 1.0), -moda)
                    rsc1 = eh_t * (moda + biasa) - et_h * (1.0 - biasa)
                    r2b = g_r2[q, dsl2]
                    r3b = g_r3[q, dsl2]
                    modb = jnp.abs(r2b)
                    biasb = jnp.maximum(jnp.minimum(r3b, 1.0), -modb)
                    rsc2 = th * (modb + biasb) - tt * (1.0 - biasb)
                    ms = ms + rsc1 * rsc1 + rsc2 * rsc2
                qo = pl.multiple_of(q * 16, 16)
                psb[pl.ds(qo, 16)] = ps
                msb[pl.ds(qo, 16)] = ms
                return carry
            lax.fori_loop(0, CH, qbody, 0)

            def gbody(g, carry):
                def kbody(k, accs):
                    ap, am = accs
                    idx = (lax.iota(jnp.int32, 16) + g * 16) * 16 + k
                    ap = ap + plsc.load_gather(psb, [idx])
                    am = am + plsc.load_gather(msb, [idx])
                    return (ap, am)
                ap, am = lax.fori_loop(
                    0, 16, kbody,
                    (jnp.zeros((16,), f32), jnp.zeros((16,), f32)))
                res = GAMMA - (0.5 * ap + _sqrt(am))
                outv[pl.ds(co + g * 16, 16)] = res
                return carry
            lax.fori_loop(0, CH // 16, gbody, 0)

        pltpu.sync_copy(outv, out.at[pl.ds(base, QPW)])

    return sc_kernel


_sc_kernel_cache = []


def _get_sc_kernel():
    if not _sc_kernel_cache:
        _sc_kernel_cache.append(_make_kernel())
    return _sc_kernel_cache[0]


@jax.jit
def kernel(heads, rels, tails, years, months, days,
           ent_embs_h, ent_embs_t, rel_embs_f, rel_embs_i, rel_embs_j,
           m_freq_h, m_freq_t, m_phi_h, m_phi_t, m_amps_h, m_amps_t,
           d_freq_h, d_freq_t, d_phi_h, d_phi_t, d_amps_h, d_amps_t,
           y_freq_h, y_freq_t, y_phi_h, y_phi_t, y_amps_h, y_amps_t):
    return _get_sc_kernel()(heads, rels, tails, years, months, days,
                      ent_embs_h, ent_embs_t, rel_embs_f, rel_embs_i, rel_embs_j,
                      m_freq_h, m_freq_t, m_phi_h, m_phi_t, m_amps_h, m_amps_t,
                      d_freq_h, d_freq_t, d_phi_h, d_phi_t, d_amps_h, d_amps_t,
                      y_freq_h, y_freq_t, y_phi_h, y_phi_t, y_amps_h, y_amps_t)
